# baseline (speedup 1.0000x reference)
import functools

import jax
import jax.numpy as jnp
from jax import lax
from jax.experimental import pallas as pl
from jax.experimental.pallas import tpu as pltpu
from jax.experimental.pallas import tpu_sc as plsc

_B, _L, _D = 4096, 200, 64
_NW = 32
_NC = 256
_TN = _NC // 128
_UNITS_N = _B // _NC
_UNITS = _L * _UNITS_N
_UNITS_W = _UNITS // _NW
_NBUF = 2
_LANE = 16


def _emb_body(idx_hbm, table_hbm, out_hbm, idx_v, g_v, t_v, gsem, wsem):
    wid = lax.axis_index("s") * 2 + lax.axis_index("c")
    u0 = wid * _UNITS_W
    iota = lax.iota(jnp.int32, _LANE)

    def _gather(u, b):
        l = u // _UNITS_N
        c = u % _UNITS_N
        pltpu.sync_copy(idx_hbm.at[l, pl.ds(c * _NC, _NC)], idx_v.at[b])
        pltpu.async_copy(table_hbm.at[idx_v.at[b]], g_v.at[b], gsem.at[b])

    def _gather_wait(b):
        pltpu.make_async_copy(
            table_hbm.at[idx_v.at[b]], g_v.at[b], gsem.at[b]
        ).wait()

    def _transpose(b):
        g2 = g_v.at[b]

        def d_body(d, carry):
            a = d // 8
            bb = d % 8
            dvec = jnp.full((_LANE,), d, dtype=jnp.int32)
            for tn in range(_TN):
                for m16 in range(128 // _LANE):
                    rows = iota + (tn * 128 + m16 * _LANE)
                    v = plsc.load_gather(g2, [rows, dvec])
                    t_v[b, a, tn, bb, pl.ds(m16 * _LANE, _LANE)] = v
            return carry

        lax.fori_loop(0, _D, d_body, 0)

    def _write(u, b):
        l = u // _UNITS_N
        c = u % _UNITS_N
        pltpu.async_copy(
            t_v.at[b],
            out_hbm.at[l, :, pl.ds(_TN * c, _TN)],
            wsem.at[b],
        )

    def _write_wait(u, b):
        l = u // _UNITS_N
        c = u % _UNITS_N
        pltpu.make_async_copy(
            t_v.at[b],
            out_hbm.at[l, :, pl.ds(_TN * c, _TN)],
            wsem.at[b],
        ).wait()

    for b in range(_NBUF):
        _gather(u0 + b, b)

    def round_body(g, carry):
        for b in range(_NBUF):
            u = u0 + g * _NBUF + b
            _gather_wait(b)
            _transpose(b)
            _write(u, b)
            _write_wait(u, b)
            _gather(u + _NBUF, b)
        return carry

    lax.fori_loop(0, _UNITS_W // _NBUF - 1, round_body, 0)

    for b in range(_NBUF):
        u = u0 + (_UNITS_W // _NBUF - 1) * _NBUF + b
        _gather_wait(b)
        _transpose(b)
        _write(u, b)
        _write_wait(u, b)


_emb = functools.partial(
    pl.kernel,
    out_type=jax.ShapeDtypeStruct((_L, _D // 8, _B // 128, 8, 128), jnp.float32),
    mesh=plsc.VectorSubcoreMesh(core_axis_name="c", subcore_axis_name="s"),
    scratch_types=[
        pltpu.VMEM((_NBUF, _NC), jnp.int32),
        pltpu.VMEM((_NBUF, _NC, _D), jnp.float32),
        pltpu.VMEM((_NBUF, _D // 8, _TN, 8, 128), jnp.float32),
        pltpu.SemaphoreType.DMA((_NBUF,)),
        pltpu.SemaphoreType.DMA((_NBUF,)),
    ],
    compiler_params=pltpu.CompilerParams(
        use_tc_tiling_on_sc=False, needs_layout_passes=False
    ),
)(_emb_body)


@jax.jit
def kernel(token_ids, weight):
    idx2d = token_ids.T
    out5 = _emb(idx2d, weight)
    out = jnp.transpose(out5, (2, 4, 0, 1, 3))
    return out.reshape(_B, _L, _D)
